# Initial kernel scaffold; baseline (speedup 1.0000x reference)
import jax
import jax.numpy as jnp
from jax.experimental import pallas as pl


def kernel(x, memory):
    raise NotImplementedError("write your pallas kernel here")


# speedup vs baseline: 2.9301x; 2.9301x over previous
import jax
import jax.numpy as jnp
from jax.experimental import pallas as pl
from jax.experimental.pallas import tpu as pltpu

N_MEM = 100000
H = 128
TOPK = 32
BLK_ROWS = 2048
N_PAD = 100352
GRID = N_PAD // BLK_ROWS
OUT_BLK = BLK_ROWS // H
NEG = -1e30


def _score_body(x_ref, mem_ref, out_ref, t_ref):
    pid = pl.program_id(0)

    @pl.when(pid == 0)
    def _():
        t_ref[...] = jnp.mean(x_ref[...], axis=0, keepdims=True)

    t = t_ref[...]
    m = mem_ref[...]
    num = jnp.sum(m * t, axis=1)
    ss = jnp.sum(m * m, axis=1)
    s = num * jax.lax.rsqrt(jnp.maximum(ss, jnp.float32(1e-16)))
    flat = pid * BLK_ROWS + jax.lax.iota(jnp.int32, BLK_ROWS)
    s = jnp.where(flat < N_MEM, s, NEG)
    out_ref[...] = s.reshape(OUT_BLK, H)


def _scores(x, memory):
    return pl.pallas_call(
        _score_body,
        grid=(GRID,),
        in_specs=[
            pl.BlockSpec((1024, H), lambda i: (0, 0)),
            pl.BlockSpec((BLK_ROWS, H), lambda i: (i, 0)),
        ],
        out_specs=pl.BlockSpec((OUT_BLK, H), lambda i: (i, 0)),
        out_shape=jax.ShapeDtypeStruct((N_PAD // H, H), jnp.float32),
        scratch_shapes=[pltpu.VMEM((1, H), jnp.float32)],
    )(x, memory)


def _select_body(scores_ref, mem_ref, out_ref, idx_ref, sem):
    s = scores_ref[...]
    rows = N_PAD // H
    r_iota = jax.lax.broadcasted_iota(jnp.int32, (rows, H), 0)
    c_iota = jax.lax.broadcasted_iota(jnp.int32, (rows, H), 1)
    flat = r_iota * H + c_iota
    big = jnp.int32(2**31 - 1)
    for k in range(TOPK):
        m = jnp.max(s)
        idx = jnp.min(jnp.where(s == m, flat, big))
        idx_ref[0, k] = idx
        s = jnp.where(flat == idx, NEG, s)
    wave = 8
    for k0 in range(0, TOPK, wave):
        copies = []
        for k in range(k0, k0 + wave):
            cp = pltpu.make_async_copy(
                mem_ref.at[pl.ds(idx_ref[0, k], 1)],
                out_ref.at[pl.ds(k, 1)], sem)
            cp.start()
            copies.append(cp)
        for cp in copies:
            cp.wait()


def _select_gather(scores, memory):
    return pl.pallas_call(
        _select_body,
        in_specs=[
            pl.BlockSpec((N_PAD // H, H), lambda: (0, 0)),
            pl.BlockSpec(memory_space=pl.ANY),
        ],
        out_specs=pl.BlockSpec((TOPK, H), lambda: (0, 0)),
        out_shape=jax.ShapeDtypeStruct((TOPK, H), jnp.float32),
        scratch_shapes=[pltpu.SMEM((1, TOPK), jnp.int32),
                        pltpu.SemaphoreType.DMA],
    )(scores, memory)


def kernel(x, memory):
    scores = _scores(x, memory)
    return _select_gather(scores, memory)
